# baseline (speedup 1.0000x reference)
import functools

import jax
import jax.numpy as jnp
from jax import lax
from jax.experimental import pallas as pl
from jax.experimental.pallas import tpu as pltpu

N_HEADS = 4
SIGMA = 6.0
ALPHA = 0.2


def _gcn_kernel(T, x_ref, w_ref, b_ref, d_ref, nbr_ref, out_ref,
                a_scr, y_scr, agg_scr):
    N, K = d_ref.shape
    NH = N * N_HEADS
    D = w_ref.shape[0]

    r_row = lax.broadcasted_iota(jnp.int32, (NH, N), 0)
    i_col = lax.broadcasted_iota(jnp.int32, (NH, N), 1)
    rep = ((r_row // N_HEADS) == i_col).astype(jnp.float32)
    d_rep = jnp.dot(rep, d_ref[...], preferred_element_type=jnp.float32)
    nbr_rep = jnp.dot(rep, nbr_ref[...].astype(jnp.float32),
                      preferred_element_type=jnp.float32)
    lam = ((lax.broadcasted_iota(jnp.int32, (NH, 1), 0) % N_HEADS) + 1
           ).astype(jnp.float32) * (1.0 / N_HEADS)
    n_f = lax.broadcasted_iota(jnp.int32, (NH, N), 1).astype(jnp.float32)
    acc = jnp.zeros((NH, N), dtype=jnp.float32)
    inv_s2 = 1.0 / (SIGMA * SIGMA)
    for k in range(K):
        wgt = jnp.exp(-(d_rep[:, k:k + 1] ** 2) * lam * inv_s2)
        acc = acc + wgt * (nbr_rep[:, k:k + 1] == n_f).astype(jnp.float32)
    a_scr[...] = acc

    x_all = x_ref[0].reshape(T * N, D)
    y_stack = lax.dot_general(x_all, w_ref[...], (((1,), (1,)), ((), ())),
                              preferred_element_type=jnp.float32)
    prev = None
    for t in range(T):
        cur = y_stack[t * N:(t + 1) * N, :]
        if t == 0:
            y_scr[:, :D] = cur
        else:
            y_scr[:, t * D:(t + 1) * D] = (1.0 - ALPHA) * cur + ALPHA * prev
        prev = cur

    agg_scr[...] = jnp.dot(a_scr[...], y_scr[...],
                           preferred_element_type=jnp.float32)

    bias = b_ref[0]
    for t in range(T):
        out_ref[0, t] = jnp.maximum(agg_scr[:, t * D:(t + 1) * D] + bias[None, :],
                                    0.0)


def kernel(x, W, b, dists, neighbors):
    B, T, N, D = x.shape
    H = N_HEADS
    NH = N * H
    b2 = b.reshape(1, D)

    body = functools.partial(_gcn_kernel, T)
    out = pl.pallas_call(
        body,
        grid=(B,),
        in_specs=[
            pl.BlockSpec((1, T, N, D), lambda c: (c, 0, 0, 0)),
            pl.BlockSpec((D, D), lambda c: (0, 0)),
            pl.BlockSpec((1, D), lambda c: (0, 0)),
            pl.BlockSpec(dists.shape, lambda c: (0, 0)),
            pl.BlockSpec(neighbors.shape, lambda c: (0, 0)),
        ],
        out_specs=pl.BlockSpec((1, T, NH, D), lambda c: (c, 0, 0, 0)),
        out_shape=jax.ShapeDtypeStruct((B, T, NH, D), jnp.float32),
        scratch_shapes=[
            pltpu.VMEM((NH, N), jnp.float32),
            pltpu.VMEM((N, T * D), jnp.float32),
            pltpu.VMEM((NH, T * D), jnp.float32),
        ],
        compiler_params=pltpu.CompilerParams(
            dimension_semantics=("parallel",),
        ),
    )(x, W, b2, dists, neighbors)
    return out.reshape(B, T, N, H, D)


# speedup vs baseline: 1.0724x; 1.0724x over previous
import functools

import jax
import jax.numpy as jnp
from jax import lax
from jax.experimental import pallas as pl
from jax.experimental.pallas import tpu as pltpu

N_HEADS = 4
SIGMA = 6.0
ALPHA = 0.2


def _gcn_kernel(T, x_ref, w_ref, b_ref, d_ref, nbr_ref, out_ref,
                a_scr, y_scr, agg_scr):
    c = pl.program_id(0)
    N, K = d_ref.shape
    NH = N * N_HEADS
    D = w_ref.shape[0]

    @pl.when(c == 0)
    def _build_a():
        r_row = lax.broadcasted_iota(jnp.int32, (NH, N), 0)
        i_col = lax.broadcasted_iota(jnp.int32, (NH, N), 1)
        rep = ((r_row // N_HEADS) == i_col).astype(jnp.float32)
        d_rep = jnp.dot(rep, d_ref[...], preferred_element_type=jnp.float32)
        nbr_rep = jnp.dot(rep, nbr_ref[...].astype(jnp.float32),
                          preferred_element_type=jnp.float32)
        lam = ((lax.broadcasted_iota(jnp.int32, (NH, 1), 0) % N_HEADS) + 1
               ).astype(jnp.float32) * (1.0 / N_HEADS)
        n_f = lax.broadcasted_iota(jnp.int32, (NH, N), 1).astype(jnp.float32)
        acc = jnp.zeros((NH, N), dtype=jnp.float32)
        inv_s2 = 1.0 / (SIGMA * SIGMA)
        for k in range(K):
            wgt = jnp.exp(-(d_rep[:, k:k + 1] ** 2) * lam * inv_s2)
            acc = acc + wgt * (nbr_rep[:, k:k + 1] == n_f).astype(jnp.float32)
        a_scr[...] = acc

    x_all = x_ref[0].reshape(T * N, D)
    y_stack = lax.dot_general(x_all, w_ref[...], (((1,), (1,)), ((), ())),
                              preferred_element_type=jnp.float32)
    prev = None
    for t in range(T):
        cur = y_stack[t * N:(t + 1) * N, :]
        if t == 0:
            y_scr[:, :D] = cur
        else:
            y_scr[:, t * D:(t + 1) * D] = (1.0 - ALPHA) * cur + ALPHA * prev
        prev = cur

    agg_scr[...] = jnp.dot(a_scr[...], y_scr[...],
                           preferred_element_type=jnp.float32)

    bias = b_ref[0]
    for t in range(T):
        out_ref[0, t] = jnp.maximum(agg_scr[:, t * D:(t + 1) * D] + bias[None, :],
                                    0.0)


def kernel(x, W, b, dists, neighbors):
    B, T, N, D = x.shape
    H = N_HEADS
    NH = N * H
    b2 = b.reshape(1, D)

    body = functools.partial(_gcn_kernel, T)
    out = pl.pallas_call(
        body,
        grid=(B,),
        in_specs=[
            pl.BlockSpec((1, T, N, D), lambda c: (c, 0, 0, 0)),
            pl.BlockSpec((D, D), lambda c: (0, 0)),
            pl.BlockSpec((1, D), lambda c: (0, 0)),
            pl.BlockSpec(dists.shape, lambda c: (0, 0)),
            pl.BlockSpec(neighbors.shape, lambda c: (0, 0)),
        ],
        out_specs=pl.BlockSpec((1, T, NH, D), lambda c: (c, 0, 0, 0)),
        out_shape=jax.ShapeDtypeStruct((B, T, NH, D), jnp.float32),
        scratch_shapes=[
            pltpu.VMEM((NH, N), jnp.float32),
            pltpu.VMEM((N, T * D), jnp.float32),
            pltpu.VMEM((NH, T * D), jnp.float32),
        ],
    )(x, W, b2, dists, neighbors)
    return out.reshape(B, T, N, H, D)
